# baseline (speedup 1.0000x reference)
import jax
import jax.numpy as jnp
import numpy as np
from jax import lax
from jax.experimental import pallas as pl
from jax.experimental.pallas import tpu as pltpu
from jax.experimental.pallas import tpu_sc as plsc

N_ATOMS = 10000
N_EDGES = 320000
R_CUT = 5.0
NC, NS = 2, 16
E_PAD = 327680
EPT = E_PAD // NS
CH = 128
NCHUNK = EPT // CH
N_PAD = 10240
APT = N_PAD // NS
RW = 128
RS = 41
PIB = "promise_in_bounds"

_SIN_C = (3.14159258, -5.16770687, 2.55003119, -0.59804419, 0.07721839)


def _muc(n_max):
    mu = np.linspace(0.0, R_CUT, n_max)
    sigma = R_CUT / float(n_max)
    return [(float(m), float(1.0 / (2.0 * sigma * sigma))) for m in mu]


def _fc16(r16):
    v = 0.5 - jnp.clip(r16 * np.float32(1.0 / R_CUT), 0.0, 1.0)
    t = v * v
    p = jnp.float32(_SIN_C[4])
    for k in (3, 2, 1, 0):
        p = p * t + jnp.float32(_SIN_C[k])
    return 0.5 * (p * v + 1.0)


def _body(shth, embh, wch, cnh, out,
          acc, sht, ev, rec, msg, cn2, wb, sem, sem2):
    c = lax.axis_index("c")
    s = lax.axis_index("s")

    z = jnp.zeros((16,), jnp.float32)

    def mrow(i, _):
        for j in range(RW // 16):
            msg[i, pl.ds(j * 16, 16)] = z
        return 0

    lax.fori_loop(0, CH, mrow, 0)
    for q in range(APT // CH):
        pltpu.sync_copy(msg, acc.at[pl.ds(s * APT + q * CH, CH)])
    pltpu.sync_copy(wch, wb)
    plsc.subcore_barrier()

    it = lax.iota(jnp.int32, 16)
    hi = it // 8
    padmask = jnp.where(it < 8, jnp.float32(1.0), jnp.float32(0.0))

    HC = CH // 2

    def stream_in(b):
        pltpu.sync_copy(shth.at[:, pl.ds(b, CH)], sht)
        pltpu.sync_copy(cnh.at[:, pl.ds(b, CH)], cn2)
        cp0 = pltpu.async_copy(
            embh.at[cn2.at[1, pl.ds(0, HC)]], ev.at[pl.ds(0, HC)], sem)
        cp1 = pltpu.async_copy(
            embh.at[cn2.at[1, pl.ds(HC, HC)]], ev.at[pl.ds(HC, HC)], sem2)
        return cp0, cp1

    def prep(mucs, sh_rows):
        @plsc.parallel_loop(0, CH, step=16, unroll=2)
        def one(i):
            ibase = (jnp.full((16,), i, jnp.int32) + it) * RS
            for m in sh_rows:
                plsc.store_scatter(rec, [ibase + m], sht[m, pl.ds(i, 16)])
            r16 = sht[16, pl.ds(i, 16)]
            fc = _fc16(r16)
            for row, (mu, c2) in enumerate(mucs):
                d = r16 - np.float32(mu)
                gq = jnp.exp(d * d * np.float32(-c2)) * fc
                plsc.store_scatter(rec, [ibase + (16 + row)], gq)

    @pl.when(c == 0)
    def _core0():
        pat0 = it % 8
        p1a = 8 + (it % 6)
        pe1 = jnp.where(it < 8, it + 16, it - 8)
        p1b = 8 + (pe1 % 6)
        p1c = 8 + ((it + 8) % 6)
        p2b = 6 + (it % 4)
        z16 = it * 0
        Wl0a = wb[pl.ds(0, 16)]
        Wl0b = wb[pl.ds(16, 16)]
        W1v0 = wb[pl.ds(32, 16)]
        W1v1 = plsc.load_gather(wb, [32 + pe1])
        W1v2 = wb[pl.ds(40, 16)]
        W1v4 = W1v1 * padmask
        W2v = wb[pl.ds(56, 16)]
        mucs = _muc(8) + _muc(6) + _muc(4)

        def chunk(g, _):
            cp0, cp1 = stream_in(s * EPT + g * CH)
            prep(mucs, range(5))

            def edge_body(e):
                eb = e * RS
                e16 = jnp.full((16,), e, jnp.int32)
                shv = rec[pl.ds(eb, 16)]
                gA = rec[pl.ds(eb + 16, 16)]
                gB = rec[pl.ds(eb + 24, 16)]
                ev0 = ev[e, pl.ds(0, 16)]
                ev1 = ev[e, pl.ds(16, 16)]
                evm = ev[e, pl.ds(8, 16)]
                em1b = plsc.load_gather(ev, [e16, pe1])
                g0 = gA.at[pat0].get(mode=PIB)
                sh0e = shv.at[z16].get(mode=PIB)
                msg[e, pl.ds(0, 16)] = sh0e * (g0 * Wl0a) * ev0
                msg[e, pl.ds(16, 16)] = sh0e * (g0 * Wl0b) * ev1
                g1a = gA.at[p1a].get(mode=PIB)
                g1b = gA.at[p1b].get(mode=PIB)
                g1c = gA.at[p1c].get(mode=PIB)
                sm0 = shv.at[z16 + 1].get(mode=PIB)
                sm1 = shv.at[z16 + 2].get(mode=PIB)
                sm2 = shv.at[z16 + 3].get(mode=PIB)
                shg = jnp.where(it < 8, sm0, sm1)
                rb0 = g1a * W1v0
                msg[e, pl.ds(32, 16)] = sm0 * rb0 * ev0
                msg[e, pl.ds(48, 16)] = shg * (g1b * W1v1) * em1b
                msg[e, pl.ds(64, 16)] = sm1 * (g1c * W1v2) * evm
                msg[e, pl.ds(80, 16)] = sm2 * rb0 * ev0
                msg[e, pl.ds(96, 16)] = sm2 * (g1b * W1v4) * em1b
                g2 = gB.at[p2b].get(mode=PIB)
                s20 = shv.at[z16 + 4].get(mode=PIB)
                msg[e, pl.ds(112, 16)] = s20 * (g2 * W2v) * ev0

            cp0.wait()
            plsc.parallel_loop(0, HC, unroll=4)(edge_body)
            cp1.wait()
            plsc.parallel_loop(HC, CH, unroll=4)(edge_body)
            pltpu.sync_copy(msg, acc.at[cn2.at[0]], add=True)
            return 0

        lax.fori_loop(0, NCHUNK, chunk, 0)

    @pl.when(c == 1)
    def _core1():
        p2 = it % 4
        p3 = 4 + (it % 2)
        pe3 = it % 8
        z16 = it * 0
        sh_r3 = [9 + jnp.minimum(2 * v + hi, 6) for v in range(4)]
        W2v = wb[pl.ds(56, 16)]
        W3v = plsc.load_gather(wb, [72 + pe3])
        W3vp = W3v * padmask
        mucs = _muc(4) + _muc(2)

        def chunk(g, _):
            cp0, cp1 = stream_in(s * EPT + g * CH)
            prep(mucs, range(5, 16))

            def edge_body(e):
                eb = e * RS
                e16 = jnp.full((16,), e, jnp.int32)
                shv = rec[pl.ds(eb, 16)]
                gA = rec[pl.ds(eb + 16, 16)]
                ev0 = ev[e, pl.ds(0, 16)]
                em3 = plsc.load_gather(ev, [e16, pe3])
                g2 = gA.at[p2].get(mode=PIB)
                rb2 = g2 * W2v
                for m in range(4):
                    sm = shv.at[z16 + 5 + m].get(mode=PIB)
                    msg[e, pl.ds(16 * m, 16)] = sm * rb2 * ev0
                g3 = gA.at[p3].get(mode=PIB)
                rb3 = g3 * W3v
                rb3p = g3 * W3vp
                for v in range(4):
                    shg = shv.at[sh_r3[v]].get(mode=PIB)
                    rb = rb3p if v == 3 else rb3
                    msg[e, pl.ds(64 + 16 * v, 16)] = shg * rb * em3

            cp0.wait()
            plsc.parallel_loop(0, HC, unroll=4)(edge_body)
            cp1.wait()
            plsc.parallel_loop(HC, CH, unroll=4)(edge_body)
            pltpu.sync_copy(msg, acc.at[cn2.at[0]], add=True)
            return 0

        lax.fori_loop(0, NCHUNK, chunk, 0)

    plsc.subcore_barrier()
    row0 = s * APT
    pltpu.sync_copy(acc.at[pl.ds(row0, APT)], out.at[c, pl.ds(row0, APT)])


@jax.jit
def _sc_call(sht, emb4, wcat, cn):
    mesh = plsc.VectorSubcoreMesh(
        core_axis_name="c", subcore_axis_name="s", num_cores=NC, num_subcores=NS)
    f = pl.kernel(
        _body,
        out_type=jax.ShapeDtypeStruct((NC, N_PAD, RW), jnp.float32),
        mesh=mesh,
        compiler_params=pltpu.CompilerParams(needs_layout_passes=False),
        scratch_types=[
            pltpu.VMEM_SHARED((N_PAD, RW), jnp.float32),
            pltpu.VMEM((17, CH), jnp.float32),
            pltpu.VMEM((CH, RW), jnp.float32),
            pltpu.VMEM((CH * RS + 16,), jnp.float32),
            pltpu.VMEM((CH, RW), jnp.float32),
            pltpu.VMEM((2, CH), jnp.int32),
            pltpu.VMEM((80,), jnp.float32),
            pltpu.SemaphoreType.DMA,
            pltpu.SemaphoreType.DMA,
        ],
    )
    return f(sht, emb4, wcat, cn)


def kernel(r, sh_0, sh_1, sh_2, sh_3, initial_center_embedding,
           W0, W1, W2, W3, centers, neighbors, n_atoms):
    E = r.shape[0]
    N = initial_center_embedding.shape[0]
    pad = E_PAD - E
    shcat = jnp.concatenate(
        [sh_0.reshape(E, 1), sh_1.reshape(E, 3), sh_2.reshape(E, 5),
         sh_3.reshape(E, 7), r.reshape(E, 1)], axis=1)
    sht = jnp.pad(shcat.T, ((0, 0), (0, pad)),
                  constant_values=0.0).at[16, E:].set(R_CUT)
    emb4 = jnp.pad(initial_center_embedding.reshape(N, 32), ((0, 0), (0, 96)))
    wcat = jnp.concatenate(
        [W0.reshape(-1), W1.reshape(-1), W2.reshape(-1), W3.reshape(-1)])
    cn = jnp.pad(jnp.stack([centers.astype(jnp.int32),
                            neighbors.astype(jnp.int32)]), ((0, 0), (0, pad)))
    out = _sc_call(sht, emb4, wcat, cn)
    b0 = out[0, :N, :32].reshape(N, 1, 32)
    b1 = out[0, :N, 32:104].reshape(N, 3, 24)
    b2 = jnp.concatenate([out[0, :N, 112:128], out[1, :N, 0:64]],
                         axis=1).reshape(N, 5, 16)
    b3 = out[1, :N, 64:120].reshape(N, 7, 8)
    return (b0, b1, b2, b3)


# speedup vs baseline: 1.0531x; 1.0531x over previous
import jax
import jax.numpy as jnp
import numpy as np
from jax import lax
from jax.experimental import pallas as pl
from jax.experimental.pallas import tpu as pltpu
from jax.experimental.pallas import tpu_sc as plsc

N_ATOMS = 10000
N_EDGES = 320000
R_CUT = 5.0
NC, NS = 2, 16
E_PAD = 327680
EPT = E_PAD // NS
CH = 128
NCHUNK = EPT // CH
N_PAD = 10240
APT = N_PAD // NS
RW = 128
RS = 41
PIB = "promise_in_bounds"

_SIN_C = (3.14159258, -5.16770687, 2.55003119, -0.59804419, 0.07721839)


def _muc(n_max):
    mu = np.linspace(0.0, R_CUT, n_max)
    sigma = R_CUT / float(n_max)
    return [(float(m), float(1.0 / (2.0 * sigma * sigma))) for m in mu]


def _fc16(r16):
    v = 0.5 - jnp.clip(r16 * np.float32(1.0 / R_CUT), 0.0, 1.0)
    t = v * v
    p = jnp.float32(_SIN_C[4])
    for k in (3, 2, 1, 0):
        p = p * t + jnp.float32(_SIN_C[k])
    return 0.5 * (p * v + 1.0)


def _body(shth, embh, wch, cnh, out,
          acc, sht, ev, rec, msg, cv, nv, wb, sem, sem2, sem3):
    c = lax.axis_index("c")
    s = lax.axis_index("s")

    z = jnp.zeros((16,), jnp.float32)

    def mrow(i, _):
        for j in range(RW // 16):
            msg[i, pl.ds(j * 16, 16)] = z
        return 0

    lax.fori_loop(0, CH, mrow, 0)
    for q in range(APT // CH):
        pltpu.sync_copy(msg, acc.at[pl.ds(s * APT + q * CH, CH)])
    pltpu.sync_copy(wch, wb)
    plsc.subcore_barrier()

    it = lax.iota(jnp.int32, 16)
    hi = it // 8
    padmask = jnp.where(it < 8, jnp.float32(1.0), jnp.float32(0.0))

    HC = CH // 2

    def stream_in(b):
        pltpu.sync_copy(shth.at[:, pl.ds(b, CH)], sht)
        pltpu.sync_copy(cnh.at[1, pl.ds(b, CH)], nv)
        cp0 = pltpu.async_copy(
            embh.at[nv.at[pl.ds(0, HC)]], ev.at[pl.ds(0, HC)], sem)
        cp1 = pltpu.async_copy(
            embh.at[nv.at[pl.ds(HC, HC)]], ev.at[pl.ds(HC, HC)], sem2)
        return cp0, cp1

    def prep(mucs, sh_rows):
        @plsc.parallel_loop(0, CH, step=16, unroll=2)
        def one(i):
            ibase = (jnp.full((16,), i, jnp.int32) + it) * RS
            for m in sh_rows:
                plsc.store_scatter(rec, [ibase + m], sht[m, pl.ds(i, 16)])
            r16 = sht[16, pl.ds(i, 16)]
            fc = _fc16(r16)
            for row, (mu, c2) in enumerate(mucs):
                d = r16 - np.float32(mu)
                gq = jnp.exp(d * d * np.float32(-c2)) * fc
                plsc.store_scatter(rec, [ibase + (16 + row)], gq)

    @pl.when(c == 0)
    def _core0():
        pat0 = it % 8
        p1a = 8 + (it % 6)
        pe1 = jnp.where(it < 8, it + 16, it - 8)
        p1b = 8 + (pe1 % 6)
        p1c = 8 + ((it + 8) % 6)
        p2b = 6 + (it % 4)
        z16 = it * 0
        Wl0a = wb[pl.ds(0, 16)]
        Wl0b = wb[pl.ds(16, 16)]
        W1v0 = wb[pl.ds(32, 16)]
        W1v1 = plsc.load_gather(wb, [32 + pe1])
        W1v2 = wb[pl.ds(40, 16)]
        W1v4 = W1v1 * padmask
        W2v = wb[pl.ds(56, 16)]
        mucs = _muc(8) + _muc(6) + _muc(4)

        def chunk(g, _):
            cp0, cp1 = stream_in(s * EPT + g * CH)
            prep(mucs, range(5))

            def edge_body(e):
                eb = e * RS
                e16 = jnp.full((16,), e, jnp.int32)
                shv = rec[pl.ds(eb, 16)]
                gA = rec[pl.ds(eb + 16, 16)]
                gB = rec[pl.ds(eb + 24, 16)]
                ev0 = ev[e, pl.ds(0, 16)]
                ev1 = ev[e, pl.ds(16, 16)]
                evm = ev[e, pl.ds(8, 16)]
                em1b = plsc.load_gather(ev, [e16, pe1])
                g0 = gA.at[pat0].get(mode=PIB)
                sh0e = shv.at[z16].get(mode=PIB)
                msg[e, pl.ds(0, 16)] = sh0e * (g0 * Wl0a) * ev0
                msg[e, pl.ds(16, 16)] = sh0e * (g0 * Wl0b) * ev1
                g1a = gA.at[p1a].get(mode=PIB)
                g1b = gA.at[p1b].get(mode=PIB)
                g1c = gA.at[p1c].get(mode=PIB)
                sm0 = shv.at[z16 + 1].get(mode=PIB)
                sm1 = shv.at[z16 + 2].get(mode=PIB)
                sm2 = shv.at[z16 + 3].get(mode=PIB)
                shg = jnp.where(it < 8, sm0, sm1)
                rb0 = g1a * W1v0
                msg[e, pl.ds(32, 16)] = sm0 * rb0 * ev0
                msg[e, pl.ds(48, 16)] = shg * (g1b * W1v1) * em1b
                msg[e, pl.ds(64, 16)] = sm1 * (g1c * W1v2) * evm
                msg[e, pl.ds(80, 16)] = sm2 * rb0 * ev0
                msg[e, pl.ds(96, 16)] = sm2 * (g1b * W1v4) * em1b
                g2 = gB.at[p2b].get(mode=PIB)
                s20 = shv.at[z16 + 4].get(mode=PIB)
                msg[e, pl.ds(112, 16)] = s20 * (g2 * W2v) * ev0

            @pl.when(g > 0)
            def _drain():
                pltpu.make_async_copy(msg, acc.at[cv], sem3).wait()

            pltpu.sync_copy(cnh.at[0, pl.ds(s * EPT + g * CH, CH)], cv)
            cp0.wait()
            plsc.parallel_loop(0, HC, unroll=4)(edge_body)
            cp1.wait()
            plsc.parallel_loop(HC, CH, unroll=4)(edge_body)
            pltpu.async_copy(msg, acc.at[cv], sem3, add=True)
            return 0

        lax.fori_loop(0, NCHUNK, chunk, 0)
        pltpu.make_async_copy(msg, acc.at[cv], sem3).wait()

    @pl.when(c == 1)
    def _core1():
        p2 = it % 4
        p3 = 4 + (it % 2)
        pe3 = it % 8
        z16 = it * 0
        sh_r3 = [9 + jnp.minimum(2 * v + hi, 6) for v in range(4)]
        W2v = wb[pl.ds(56, 16)]
        W3v = plsc.load_gather(wb, [72 + pe3])
        W3vp = W3v * padmask
        mucs = _muc(4) + _muc(2)

        def chunk(g, _):
            cp0, cp1 = stream_in(s * EPT + g * CH)
            prep(mucs, range(5, 16))

            def edge_body(e):
                eb = e * RS
                e16 = jnp.full((16,), e, jnp.int32)
                shv = rec[pl.ds(eb, 16)]
                gA = rec[pl.ds(eb + 16, 16)]
                ev0 = ev[e, pl.ds(0, 16)]
                em3 = plsc.load_gather(ev, [e16, pe3])
                g2 = gA.at[p2].get(mode=PIB)
                rb2 = g2 * W2v
                for m in range(4):
                    sm = shv.at[z16 + 5 + m].get(mode=PIB)
                    msg[e, pl.ds(16 * m, 16)] = sm * rb2 * ev0
                g3 = gA.at[p3].get(mode=PIB)
                rb3 = g3 * W3v
                rb3p = g3 * W3vp
                for v in range(4):
                    shg = shv.at[sh_r3[v]].get(mode=PIB)
                    rb = rb3p if v == 3 else rb3
                    msg[e, pl.ds(64 + 16 * v, 16)] = shg * rb * em3

            @pl.when(g > 0)
            def _drain():
                pltpu.make_async_copy(msg, acc.at[cv], sem3).wait()

            pltpu.sync_copy(cnh.at[0, pl.ds(s * EPT + g * CH, CH)], cv)
            cp0.wait()
            plsc.parallel_loop(0, HC, unroll=4)(edge_body)
            cp1.wait()
            plsc.parallel_loop(HC, CH, unroll=4)(edge_body)
            pltpu.async_copy(msg, acc.at[cv], sem3, add=True)
            return 0

        lax.fori_loop(0, NCHUNK, chunk, 0)
        pltpu.make_async_copy(msg, acc.at[cv], sem3).wait()

    plsc.subcore_barrier()
    row0 = s * APT
    pltpu.sync_copy(acc.at[pl.ds(row0, APT)], out.at[c, pl.ds(row0, APT)])


@jax.jit
def _sc_call(sht, emb4, wcat, cn):
    mesh = plsc.VectorSubcoreMesh(
        core_axis_name="c", subcore_axis_name="s", num_cores=NC, num_subcores=NS)
    f = pl.kernel(
        _body,
        out_type=jax.ShapeDtypeStruct((NC, N_PAD, RW), jnp.float32),
        mesh=mesh,
        compiler_params=pltpu.CompilerParams(needs_layout_passes=False),
        scratch_types=[
            pltpu.VMEM_SHARED((N_PAD, RW), jnp.float32),
            pltpu.VMEM((17, CH), jnp.float32),
            pltpu.VMEM((CH, RW), jnp.float32),
            pltpu.VMEM((CH * RS + 16,), jnp.float32),
            pltpu.VMEM((CH, RW), jnp.float32),
            pltpu.VMEM((CH,), jnp.int32),
            pltpu.VMEM((CH,), jnp.int32),
            pltpu.VMEM((80,), jnp.float32),
            pltpu.SemaphoreType.DMA,
            pltpu.SemaphoreType.DMA,
            pltpu.SemaphoreType.DMA,
        ],
    )
    return f(sht, emb4, wcat, cn)


def kernel(r, sh_0, sh_1, sh_2, sh_3, initial_center_embedding,
           W0, W1, W2, W3, centers, neighbors, n_atoms):
    E = r.shape[0]
    N = initial_center_embedding.shape[0]
    pad = E_PAD - E
    shcat = jnp.concatenate(
        [sh_0.reshape(E, 1), sh_1.reshape(E, 3), sh_2.reshape(E, 5),
         sh_3.reshape(E, 7), r.reshape(E, 1)], axis=1)
    sht = jnp.pad(shcat.T, ((0, 0), (0, pad)),
                  constant_values=0.0).at[16, E:].set(R_CUT)
    emb4 = jnp.pad(initial_center_embedding.reshape(N, 32), ((0, 0), (0, 96)))
    wcat = jnp.concatenate(
        [W0.reshape(-1), W1.reshape(-1), W2.reshape(-1), W3.reshape(-1)])
    cn = jnp.pad(jnp.stack([centers.astype(jnp.int32),
                            neighbors.astype(jnp.int32)]), ((0, 0), (0, pad)))
    out = _sc_call(sht, emb4, wcat, cn)
    b0 = out[0, :N, :32].reshape(N, 1, 32)
    b1 = out[0, :N, 32:104].reshape(N, 3, 24)
    b2 = jnp.concatenate([out[0, :N, 112:128], out[1, :N, 0:64]],
                         axis=1).reshape(N, 5, 16)
    b3 = out[1, :N, 64:120].reshape(N, 7, 8)
    return (b0, b1, b2, b3)
